# baseline (speedup 1.0000x reference)
import functools

import jax
import jax.numpy as jnp
from jax import lax
from jax.experimental import pallas as pl
from jax.experimental.pallas import tpu as pltpu
from jax.experimental.pallas import tpu_sc as plsc

_N = 262144
_K = 64
_NC = 2
_NS = 16
_NW = _NC * _NS

_N_TC = 32768
_N_SC = _N - _N_TC
_PER_W = _N_SC // _NW

_BSTEPS = 8
_BROWS = (_N // 128) // _BSTEPS
_DW = _N_TC // _BSTEPS


def _sc_colsums(flat_dist, assignments):
    mesh = plsc.VectorSubcoreMesh(core_axis_name="c", subcore_axis_name="s")

    @functools.partial(
        pl.kernel,
        mesh=mesh,
        out_type=jax.ShapeDtypeStruct((_NW * 128,), jnp.float32),
        scratch_types=[
            pltpu.VMEM((_PER_W,), jnp.int32),
            pltpu.VMEM((_PER_W,), jnp.int32),
            pltpu.VMEM((_PER_W,), jnp.float32),
            pltpu.VMEM((128,), jnp.float32),
        ] + [pltpu.SemaphoreType.DMA] * 10,
    )
    def sc_kernel(dist_hbm, a_hbm, cs_out,
                  a_v, idx_v, val_v, acc_v, *sems):
        sid = lax.axis_index("s")
        wid = sid * _NC + lax.axis_index("c")
        base = wid * _PER_W

        stg0 = pltpu.async_copy(a_hbm.at[pl.ds(base, 2048)],
                                a_v.at[pl.ds(0, 2048)], sems[8])
        stg1 = pltpu.async_copy(a_hbm.at[pl.ds(base + 2048, _PER_W - 2048)],
                                a_v.at[pl.ds(2048, _PER_W - 2048)], sems[9])

        lane = lax.iota(jnp.int32, 16)

        def build(g2, carry):
            g = g2 * 2
            a0 = a_v[pl.ds(g * 16, 16)]
            a1 = a_v[pl.ds(g * 16 + 16, 16)]
            s = base + g * 16
            c0 = ((s >> 7) * 1024 + (s & 127)) + lane
            c1 = (((s + 16) >> 7) * 1024 + ((s + 16) & 127)) + lane
            p0 = (a0 << 7) + (a0 >> 3) * 2096128 + c0
            p1 = (a1 << 7) + (a1 >> 3) * 2096128 + c1
            idx_v[pl.ds(g * 16, 16)] = p0
            idx_v[pl.ds(g * 16 + 16, 16)] = p1
            return carry

        bounds = (0, 256, 1280, 3328, _PER_W)
        cps = []
        for k in range(4):
            if k == 0:
                stg0.wait()
            elif k == 2:
                stg1.wait()
            lax.fori_loop(bounds[k] // 32, bounds[k + 1] // 32, build, 0,
                          unroll=8)
            n = bounds[k + 1] - bounds[k]
            cps.append(pltpu.async_copy(
                dist_hbm.at[idx_v.at[pl.ds(bounds[k], n)]],
                val_v.at[pl.ds(bounds[k], n)], sems[k]))

        zf = jnp.zeros((16,), jnp.float32)

        def accum(g, carry):
            c0, c1, c2, c3 = carry
            b = g * 64
            c0 = c0 + val_v[pl.ds(b, 16)]
            c1 = c1 + val_v[pl.ds(b + 16, 16)]
            c2 = c2 + val_v[pl.ds(b + 32, 16)]
            c3 = c3 + val_v[pl.ds(b + 48, 16)]
            return (c0, c1, c2, c3)

        acc = (zf, zf, zf, zf)
        for k in range(4):
            cps[k].wait()
            acc = lax.fori_loop(bounds[k] // 64, bounds[k + 1] // 64,
                                accum, acc, unroll=4)

        for c in range(4):
            acc_v[pl.ds(c * 16, 16)] = acc[c]
            acc_v[pl.ds(64 + c * 16, 16)] = zf
        pltpu.sync_copy(acc_v, cs_out.at[pl.ds(wid * 128, 128)])

    return sc_kernel(flat_dist, assignments)


def _bincount_body(a_ref, ad_ref, d_ref, cnt_out, cs_out, acc_ref, accd_ref):
    g = pl.program_id(0)

    @pl.when(g == 0)
    def _init():
        acc_ref[...] = jnp.zeros((_K, 128), jnp.int32)
        accd_ref[...] = jnp.zeros((_K, 128), jnp.float32)

    row_iota = lax.broadcasted_iota(jnp.int32, (_K, 128), 0)

    def body(r, acc):
        arow = a_ref[pl.ds(r, 1), :]
        return acc + (row_iota == arow).astype(jnp.int32)

    acc_ref[...] = lax.fori_loop(0, _BROWS, body, acc_ref[...], unroll=8)

    ad = ad_ref[...]
    accd = accd_ref[...]
    for r in range(_DW // 128):
        arow = ad[r:r + 1, :]
        dsub = d_ref[:, r * 128:(r + 1) * 128]
        mask = row_iota == arow
        accd = accd + jnp.where(mask, dsub, 0.0)
    accd_ref[...] = accd

    @pl.when(g == _BSTEPS - 1)
    def _fin():
        cnt_out[...] = acc_ref[...]
        s1 = jnp.sum(accd_ref[...], axis=0, keepdims=True)
        cs_out[...] = s1[:, 0:_K] + s1[:, _K:2 * _K]


def _combine_body(cs_ref, cst_ref, cnt_ref, out_ref):
    cs = (jnp.sum(cs_ref[...], axis=0, keepdims=True)[:, 0:_K]
          + cst_ref[...])
    cnt = jnp.sum(cnt_ref[...], axis=1, keepdims=True)
    valid = cnt > 0
    cntf = jnp.maximum(cnt, 1).astype(jnp.float32)
    recip = jnp.where(valid, 1.0 / cntf, 0.0)
    total = jax.lax.dot_general(
        cs, recip, (((1,), (0,)), ((), ())),
        precision=jax.lax.Precision.HIGHEST,
        preferred_element_type=jnp.float32)
    n_valid = jnp.sum(valid.astype(jnp.float32))
    out_ref[...] = total / jnp.maximum(n_valid, 1.0)


def kernel(distances, assignments):
    flat = (distances.T.reshape(8, 8, 2048, 128)
            .transpose(0, 2, 1, 3).reshape(-1))
    dt = distances.T
    a2 = assignments.reshape(_N // 128, 128)
    cs = _sc_colsums(flat, assignments)
    tail0 = _N_SC // _DW
    cnt, cs_tail = pl.pallas_call(
        _bincount_body,
        grid=(_BSTEPS,),
        in_specs=[
            pl.BlockSpec((_BROWS, 128), lambda g: (g, 0)),
            pl.BlockSpec((_DW // 128, 128), lambda g: (tail0 + g, 0)),
            pl.BlockSpec((_K, _DW), lambda g: (0, tail0 + g)),
        ],
        out_specs=[
            pl.BlockSpec((_K, 128), lambda g: (0, 0)),
            pl.BlockSpec((1, _K), lambda g: (0, 0)),
        ],
        out_shape=(
            jax.ShapeDtypeStruct((_K, 128), jnp.int32),
            jax.ShapeDtypeStruct((1, _K), jnp.float32),
        ),
        scratch_shapes=[pltpu.VMEM((_K, 128), jnp.int32),
                        pltpu.VMEM((_K, 128), jnp.float32)],
    )(a2, a2, dt)
    out = pl.pallas_call(
        _combine_body,
        out_shape=jax.ShapeDtypeStruct((1, 1), jnp.float32),
    )(cs.reshape(_NW, 128), cs_tail, cnt)
    return out[0, 0]


# speedup vs baseline: 1.0072x; 1.0072x over previous
import functools

import jax
import jax.numpy as jnp
from jax import lax
from jax.experimental import pallas as pl
from jax.experimental.pallas import tpu as pltpu
from jax.experimental.pallas import tpu_sc as plsc

_N = 262144
_K = 64
_NC = 2
_NS = 16
_NW = _NC * _NS

_N_TC = 32768
_N_SC = _N - _N_TC
_PER_W = _N_SC // _NW

_BSTEPS = 8
_BROWS = (_N // 128) // _BSTEPS
_DW = _N_TC // _BSTEPS


def _sc_colsums(flat_dist, assignments):
    mesh = plsc.VectorSubcoreMesh(core_axis_name="c", subcore_axis_name="s")

    @functools.partial(
        pl.kernel,
        mesh=mesh,
        out_type=jax.ShapeDtypeStruct((_NW * 128,), jnp.float32),
        scratch_types=[
            pltpu.VMEM((_PER_W,), jnp.int32),
            pltpu.VMEM((_PER_W,), jnp.int32),
            pltpu.VMEM((_PER_W,), jnp.float32),
            pltpu.VMEM((128,), jnp.float32),
        ] + [pltpu.SemaphoreType.DMA] * 10,
    )
    def sc_kernel(dist_hbm, a_hbm, cs_out,
                  a_v, idx_v, val_v, acc_v, *sems):
        sid = lax.axis_index("s")
        wid = sid * _NC + lax.axis_index("c")
        base = wid * _PER_W

        stg0 = pltpu.async_copy(a_hbm.at[pl.ds(base, 2048)],
                                a_v.at[pl.ds(0, 2048)], sems[8])
        stg1 = pltpu.async_copy(a_hbm.at[pl.ds(base + 2048, _PER_W - 2048)],
                                a_v.at[pl.ds(2048, _PER_W - 2048)], sems[9])

        lane = lax.iota(jnp.int32, 16)

        def build(g2, carry):
            g = g2 * 2
            a0 = a_v[pl.ds(g * 16, 16)]
            a1 = a_v[pl.ds(g * 16 + 16, 16)]
            s = base + g * 16
            c0 = ((s >> 7) * 1024 + (s & 127)) + lane
            c1 = (((s + 16) >> 7) * 1024 + ((s + 16) & 127)) + lane
            p0 = (a0 << 7) + (a0 >> 3) * 2096128 + c0
            p1 = (a1 << 7) + (a1 >> 3) * 2096128 + c1
            idx_v[pl.ds(g * 16, 16)] = p0
            idx_v[pl.ds(g * 16 + 16, 16)] = p1
            return carry

        bounds = (0, 512, 2048, 4096, _PER_W)
        cps = []
        for k in range(4):
            if k == 0:
                stg0.wait()
            elif k == 2:
                stg1.wait()
            lax.fori_loop(bounds[k] // 32, bounds[k + 1] // 32, build, 0,
                          unroll=8)
            n = bounds[k + 1] - bounds[k]
            cps.append(pltpu.async_copy(
                dist_hbm.at[idx_v.at[pl.ds(bounds[k], n)]],
                val_v.at[pl.ds(bounds[k], n)], sems[k]))

        zf = jnp.zeros((16,), jnp.float32)

        def accum(g, carry):
            c0, c1, c2, c3 = carry
            b = g * 64
            c0 = c0 + val_v[pl.ds(b, 16)]
            c1 = c1 + val_v[pl.ds(b + 16, 16)]
            c2 = c2 + val_v[pl.ds(b + 32, 16)]
            c3 = c3 + val_v[pl.ds(b + 48, 16)]
            return (c0, c1, c2, c3)

        acc = (zf, zf, zf, zf)
        for k in range(4):
            cps[k].wait()
            acc = lax.fori_loop(bounds[k] // 64, bounds[k + 1] // 64,
                                accum, acc, unroll=4)

        for c in range(4):
            acc_v[pl.ds(c * 16, 16)] = acc[c]
            acc_v[pl.ds(64 + c * 16, 16)] = zf
        pltpu.sync_copy(acc_v, cs_out.at[pl.ds(wid * 128, 128)])

    return sc_kernel(flat_dist, assignments)


def _bincount_body(a_ref, ad_ref, d_ref, cnt_out, cs_out,
                   acc_ref, accd_ref, a16_ref):
    g = pl.program_id(0)

    @pl.when(g == 0)
    def _init():
        acc_ref[...] = jnp.zeros((_K, 128), jnp.int16)
        accd_ref[...] = jnp.zeros((_K, 128), jnp.float32)

    a16_ref[...] = a_ref[...].astype(jnp.int16)
    row_iota16 = lax.broadcasted_iota(jnp.int16, (_K, 128), 0)

    def body(r, acc):
        blk = a16_ref[pl.ds(pl.multiple_of(r * 16, 16), 16), :]
        for k in range(16):
            acc = acc + (row_iota16 == blk[k:k + 1, :]).astype(jnp.int16)
        return acc

    acc_ref[...] = lax.fori_loop(0, _BROWS // 16, body, acc_ref[...])

    row_iota = lax.broadcasted_iota(jnp.int32, (_K, 128), 0)

    ad = ad_ref[...]
    accd = accd_ref[...]
    for r in range(_DW // 128):
        arow = ad[r:r + 1, :]
        dsub = d_ref[:, r * 128:(r + 1) * 128]
        mask = row_iota == arow
        accd = accd + jnp.where(mask, dsub, 0.0)
    accd_ref[...] = accd

    @pl.when(g == _BSTEPS - 1)
    def _fin():
        cnt_out[...] = acc_ref[...].astype(jnp.int32)
        s1 = jnp.sum(accd_ref[...], axis=0, keepdims=True)
        cs_out[...] = s1[:, 0:_K] + s1[:, _K:2 * _K]


def _combine_body(cs_ref, cst_ref, cnt_ref, out_ref):
    cs = (jnp.sum(cs_ref[...], axis=0, keepdims=True)[:, 0:_K]
          + cst_ref[...])
    cnt = jnp.sum(cnt_ref[...], axis=1, keepdims=True)
    valid = cnt > 0
    cntf = jnp.maximum(cnt, 1).astype(jnp.float32)
    recip = jnp.where(valid, 1.0 / cntf, 0.0)
    total = jax.lax.dot_general(
        cs, recip, (((1,), (0,)), ((), ())),
        precision=jax.lax.Precision.HIGHEST,
        preferred_element_type=jnp.float32)
    n_valid = jnp.sum(valid.astype(jnp.float32))
    out_ref[...] = total / jnp.maximum(n_valid, 1.0)


def kernel(distances, assignments):
    flat = (distances.T.reshape(8, 8, 2048, 128)
            .transpose(0, 2, 1, 3).reshape(-1))
    dt = distances.T
    a2 = assignments.reshape(_N // 128, 128)
    cs = _sc_colsums(flat, assignments)
    tail0 = _N_SC // _DW
    cnt, cs_tail = pl.pallas_call(
        _bincount_body,
        grid=(_BSTEPS,),
        in_specs=[
            pl.BlockSpec((_BROWS, 128), lambda g: (g, 0)),
            pl.BlockSpec((_DW // 128, 128), lambda g: (tail0 + g, 0)),
            pl.BlockSpec((_K, _DW), lambda g: (0, tail0 + g)),
        ],
        out_specs=[
            pl.BlockSpec((_K, 128), lambda g: (0, 0)),
            pl.BlockSpec((1, _K), lambda g: (0, 0)),
        ],
        out_shape=(
            jax.ShapeDtypeStruct((_K, 128), jnp.int32),
            jax.ShapeDtypeStruct((1, _K), jnp.float32),
        ),
        scratch_shapes=[pltpu.VMEM((_K, 128), jnp.int16),
                        pltpu.VMEM((_K, 128), jnp.float32),
                        pltpu.VMEM((_BROWS, 128), jnp.int16)],
    )(a2, a2, dt)
    out = pl.pallas_call(
        _combine_body,
        out_shape=jax.ShapeDtypeStruct((1, 1), jnp.float32),
    )(cs.reshape(_NW, 128), cs_tail, cnt)
    return out[0, 0]


# speedup vs baseline: 1.0171x; 1.0098x over previous
import functools

import jax
import jax.numpy as jnp
from jax import lax
from jax.experimental import pallas as pl
from jax.experimental.pallas import tpu as pltpu
from jax.experimental.pallas import tpu_sc as plsc

_N = 262144
_K = 64
_NC = 2
_NS = 16
_NW = _NC * _NS

_N_TC = 65536
_N_SC = _N - _N_TC
_PER_W = _N_SC // _NW

_BSTEPS = 8
_BROWS = (_N // 128) // _BSTEPS
_DW = _N_TC // _BSTEPS


def _sc_colsums(flat_dist, assignments):
    mesh = plsc.VectorSubcoreMesh(core_axis_name="c", subcore_axis_name="s")

    @functools.partial(
        pl.kernel,
        mesh=mesh,
        out_type=jax.ShapeDtypeStruct((_NW * 128,), jnp.float32),
        scratch_types=[
            pltpu.VMEM((_PER_W,), jnp.int32),
            pltpu.VMEM((_PER_W,), jnp.int32),
            pltpu.VMEM((_PER_W,), jnp.float32),
            pltpu.VMEM((128,), jnp.float32),
        ] + [pltpu.SemaphoreType.DMA] * 10,
    )
    def sc_kernel(dist_hbm, a_hbm, cs_out,
                  a_v, idx_v, val_v, acc_v, *sems):
        sid = lax.axis_index("s")
        wid = sid * _NC + lax.axis_index("c")
        base = wid * _PER_W

        stg0 = pltpu.async_copy(a_hbm.at[pl.ds(base, 2048)],
                                a_v.at[pl.ds(0, 2048)], sems[8])
        stg1 = pltpu.async_copy(a_hbm.at[pl.ds(base + 2048, _PER_W - 2048)],
                                a_v.at[pl.ds(2048, _PER_W - 2048)], sems[9])

        lane = lax.iota(jnp.int32, 16)

        def build(g2, carry):
            g = g2 * 2
            a0 = a_v[pl.ds(g * 16, 16)]
            a1 = a_v[pl.ds(g * 16 + 16, 16)]
            s = base + g * 16
            c0 = ((s >> 7) * 1024 + (s & 127)) + lane
            c1 = (((s + 16) >> 7) * 1024 + ((s + 16) & 127)) + lane
            p0 = (a0 << 7) + (a0 >> 3) * 2096128 + c0
            p1 = (a1 << 7) + (a1 >> 3) * 2096128 + c1
            idx_v[pl.ds(g * 16, 16)] = p0
            idx_v[pl.ds(g * 16 + 16, 16)] = p1
            return carry

        bounds = (0, 512, 2048, 4096, _PER_W)
        cps = []
        for k in range(4):
            if k == 0:
                stg0.wait()
            elif k == 2:
                stg1.wait()
            lax.fori_loop(bounds[k] // 32, bounds[k + 1] // 32, build, 0,
                          unroll=8)
            n = bounds[k + 1] - bounds[k]
            cps.append(pltpu.async_copy(
                dist_hbm.at[idx_v.at[pl.ds(bounds[k], n)]],
                val_v.at[pl.ds(bounds[k], n)], sems[k]))

        zf = jnp.zeros((16,), jnp.float32)

        def accum(g, carry):
            c0, c1, c2, c3 = carry
            b = g * 64
            c0 = c0 + val_v[pl.ds(b, 16)]
            c1 = c1 + val_v[pl.ds(b + 16, 16)]
            c2 = c2 + val_v[pl.ds(b + 32, 16)]
            c3 = c3 + val_v[pl.ds(b + 48, 16)]
            return (c0, c1, c2, c3)

        acc = (zf, zf, zf, zf)
        for k in range(4):
            cps[k].wait()
            acc = lax.fori_loop(bounds[k] // 64, bounds[k + 1] // 64,
                                accum, acc, unroll=4)

        for c in range(4):
            acc_v[pl.ds(c * 16, 16)] = acc[c]
            acc_v[pl.ds(64 + c * 16, 16)] = zf
        pltpu.sync_copy(acc_v, cs_out.at[pl.ds(wid * 128, 128)])

    return sc_kernel(flat_dist, assignments)


def _bincount_body(a_ref, ad_ref, d_ref, cnt_out, cs_out,
                   acc_ref, accd_ref, a16_ref):
    g = pl.program_id(0)

    @pl.when(g == 0)
    def _init():
        acc_ref[...] = jnp.zeros((_K, 128), jnp.int16)
        accd_ref[...] = jnp.zeros((_K, 128), jnp.float32)

    a16_ref[...] = a_ref[...].astype(jnp.int16)
    row_iota16 = lax.broadcasted_iota(jnp.int16, (_K, 128), 0)

    def body(r, acc):
        blk = a16_ref[pl.ds(pl.multiple_of(r * 16, 16), 16), :]
        for k in range(16):
            acc = acc + (row_iota16 == blk[k:k + 1, :]).astype(jnp.int16)
        return acc

    acc_ref[...] = lax.fori_loop(0, _BROWS // 16, body, acc_ref[...])

    row_iota = lax.broadcasted_iota(jnp.int32, (_K, 128), 0)

    ad = ad_ref[...]
    accd = accd_ref[...]
    for r in range(_DW // 128):
        arow = ad[r:r + 1, :]
        dsub = d_ref[:, r * 128:(r + 1) * 128]
        mask = row_iota == arow
        accd = accd + jnp.where(mask, dsub, 0.0)
    accd_ref[...] = accd

    @pl.when(g == _BSTEPS - 1)
    def _fin():
        cnt_out[...] = acc_ref[...].astype(jnp.int32)
        s1 = jnp.sum(accd_ref[...], axis=0, keepdims=True)
        cs_out[...] = s1[:, 0:_K] + s1[:, _K:2 * _K]


def _combine_body(cs_ref, cst_ref, cnt_ref, out_ref):
    cs = (jnp.sum(cs_ref[...], axis=0, keepdims=True)[:, 0:_K]
          + cst_ref[...])
    cnt = jnp.sum(cnt_ref[...], axis=1, keepdims=True)
    valid = cnt > 0
    cntf = jnp.maximum(cnt, 1).astype(jnp.float32)
    recip = jnp.where(valid, 1.0 / cntf, 0.0)
    total = jax.lax.dot_general(
        cs, recip, (((1,), (0,)), ((), ())),
        precision=jax.lax.Precision.HIGHEST,
        preferred_element_type=jnp.float32)
    n_valid = jnp.sum(valid.astype(jnp.float32))
    out_ref[...] = total / jnp.maximum(n_valid, 1.0)


def kernel(distances, assignments):
    flat = (distances.T.reshape(8, 8, 2048, 128)
            .transpose(0, 2, 1, 3).reshape(-1))
    dt = distances.T
    a2 = assignments.reshape(_N // 128, 128)
    cs = _sc_colsums(flat, assignments)
    tail0 = _N_SC // _DW
    cnt, cs_tail = pl.pallas_call(
        _bincount_body,
        grid=(_BSTEPS,),
        in_specs=[
            pl.BlockSpec((_BROWS, 128), lambda g: (g, 0)),
            pl.BlockSpec((_DW // 128, 128), lambda g: (tail0 + g, 0)),
            pl.BlockSpec((_K, _DW), lambda g: (0, tail0 + g)),
        ],
        out_specs=[
            pl.BlockSpec((_K, 128), lambda g: (0, 0)),
            pl.BlockSpec((1, _K), lambda g: (0, 0)),
        ],
        out_shape=(
            jax.ShapeDtypeStruct((_K, 128), jnp.int32),
            jax.ShapeDtypeStruct((1, _K), jnp.float32),
        ),
        scratch_shapes=[pltpu.VMEM((_K, 128), jnp.int16),
                        pltpu.VMEM((_K, 128), jnp.float32),
                        pltpu.VMEM((_BROWS, 128), jnp.int16)],
    )(a2, a2, dt)
    out = pl.pallas_call(
        _combine_body,
        out_shape=jax.ShapeDtypeStruct((1, 1), jnp.float32),
    )(cs.reshape(_NW, 128), cs_tail, cnt)
    return out[0, 0]
